# baseline (speedup 1.0000x reference)
import jax
import jax.numpy as jnp
from jax import lax
from jax.experimental import pallas as pl
from jax.experimental.pallas import tpu as pltpu
from jax.experimental.pallas import tpu_sc as plsc

BATCH = 16384
EMBED_DIM = 32

_NC = 2
_NS = 16
_NW = _NC * _NS
_B_PER_W = BATCH // _NW
_CHUNK = 128
_NCHUNK = _B_PER_W // _CHUNK


def _sc_gather_body(uids_hbm, iids_hbm, utab_hbm, itab_hbm,
                    u_out, i_out,
                    uidx_v, iidx_v, urows_v, irows_v, sem):
    wid = lax.axis_index("s") * _NC + lax.axis_index("c")
    base = wid * _B_PER_W
    pltpu.sync_copy(uids_hbm.at[pl.ds(base, _B_PER_W)], uidx_v)
    pltpu.sync_copy(iids_hbm.at[pl.ds(base, _B_PER_W)], iidx_v)
    copies = []
    for c in range(_NCHUNK):
        sl = pl.ds(c * _CHUNK, _CHUNK)
        copies.append(
            pltpu.async_copy(utab_hbm.at[uidx_v.at[sl]], urows_v.at[sl], sem))
        copies.append(
            pltpu.async_copy(itab_hbm.at[iidx_v.at[sl]], irows_v.at[sl], sem))
    for cp in copies:
        cp.wait()
    pltpu.sync_copy(urows_v, u_out.at[pl.ds(base, _B_PER_W)])
    pltpu.sync_copy(irows_v, i_out.at[pl.ds(base, _B_PER_W)])


def _sc_gather(user_ids, item_ids, user_table, item_table):
    mesh = plsc.VectorSubcoreMesh(
        core_axis_name="c", subcore_axis_name="s",
        num_cores=_NC, num_subcores=_NS)
    f = pl.kernel(
        _sc_gather_body,
        out_type=(
            jax.ShapeDtypeStruct((BATCH, EMBED_DIM), jnp.bfloat16),
            jax.ShapeDtypeStruct((BATCH, EMBED_DIM), jnp.bfloat16),
        ),
        mesh=mesh,
        scratch_types=(
            pltpu.VMEM((_B_PER_W,), jnp.int32),
            pltpu.VMEM((_B_PER_W,), jnp.int32),
            pltpu.VMEM((_B_PER_W, EMBED_DIM), jnp.bfloat16),
            pltpu.VMEM((_B_PER_W, EMBED_DIM), jnp.bfloat16),
            pltpu.SemaphoreType.DMA,
        ),
        compiler_params=pltpu.CompilerParams(use_tc_tiling_on_sc=False),
    )
    return f(user_ids, item_ids, user_table, item_table)


_MLP_BLK = 2048


def _mlp_body(u_ref, i_ref, w1a_ref, w1b_ref, b1_ref, w2_ref, b2_ref,
              w3_ref, b3_ref, w4_ref, b4_ref, out_ref):
    f32 = jnp.float32
    u = u_ref[...].astype(f32)
    i = i_ref[...].astype(f32)
    h = (jnp.dot(u, w1a_ref[...], preferred_element_type=f32)
         + jnp.dot(i, w1b_ref[...], preferred_element_type=f32)
         + b1_ref[...])
    h = jnp.maximum(h, 0.0)
    h = jnp.dot(h, w2_ref[...], preferred_element_type=f32) + b2_ref[...]
    h = jnp.maximum(h, 0.0)
    h = jnp.dot(h, w3_ref[...], preferred_element_type=f32) + b3_ref[...]
    h = jnp.maximum(h, 0.0)
    z = jnp.dot(h, w4_ref[...], preferred_element_type=f32) + b4_ref[...]
    out_ref[...] = jax.nn.sigmoid(z)


def _mlp(u, i, W1, b1, W2, b2, W3, b3, W4, b4):
    w1a = W1[:EMBED_DIM]
    w1b = W1[EMBED_DIM:]
    grid = BATCH // _MLP_BLK
    full = lambda a: pl.BlockSpec(a.shape, lambda b: (0,) * a.ndim)
    out = pl.pallas_call(
        _mlp_body,
        grid=(grid,),
        in_specs=[
            pl.BlockSpec((_MLP_BLK, EMBED_DIM), lambda b: (b, 0)),
            pl.BlockSpec((_MLP_BLK, EMBED_DIM), lambda b: (b, 0)),
            full(w1a), full(w1b),
            pl.BlockSpec((1, 64), lambda b: (0, 0)),
            full(W2),
            pl.BlockSpec((1, 32), lambda b: (0, 0)),
            full(W3),
            pl.BlockSpec((1, 16), lambda b: (0, 0)),
            full(W4),
            pl.BlockSpec((1, 1), lambda b: (0, 0)),
        ],
        out_specs=pl.BlockSpec((_MLP_BLK, 1), lambda b: (b, 0)),
        out_shape=jax.ShapeDtypeStruct((BATCH, 1), jnp.float32),
    )(u, i, w1a, w1b, b1.reshape(1, 64), W2, b2.reshape(1, 32),
      W3, b3.reshape(1, 16), W4, b4.reshape(1, 1))
    return out[:, 0]


def kernel(user_ids, item_ids, user_table, item_table,
           W1, b1, W2, b2, W3, b3, W4, b4):
    utab = user_table.astype(jnp.bfloat16)
    itab = item_table.astype(jnp.bfloat16)
    u, i = _sc_gather(user_ids, item_ids, utab, itab)
    return _mlp(u, i, W1, b1, W2, b2, W3, b3, W4, b4)


# speedup vs baseline: 1.1213x; 1.1213x over previous
import jax
import jax.numpy as jnp
from jax import lax
from jax.experimental import pallas as pl
from jax.experimental.pallas import tpu as pltpu
from jax.experimental.pallas import tpu_sc as plsc

BATCH = 16384
EMBED_DIM = 32
_GRP = 4
_GW = _GRP * EMBED_DIM

_NC = 2
_NS = 16
_NW = _NC * _NS
_B_PER_W = BATCH // _NW
_CHUNK = 128
_NCHUNK = _B_PER_W // _CHUNK


def _sc_gather_body(uids_hbm, iids_hbm, utab_hbm, itab_hbm,
                    u_out, i_out, uq_v, iq_v, ubuf, ibuf, sem):
    wid = lax.axis_index("s") * _NC + lax.axis_index("c")
    base = wid * _B_PER_W
    pltpu.sync_copy(uids_hbm.at[pl.ds(base, _B_PER_W)], uq_v)
    pltpu.sync_copy(iids_hbm.at[pl.ds(base, _B_PER_W)], iq_v)

    def shift(g, carry):
        sl = pl.ds(g * 16, 16)
        uq_v[sl] = lax.shift_right_logical(uq_v[sl], 2)
        iq_v[sl] = lax.shift_right_logical(iq_v[sl], 2)
        return carry
    lax.fori_loop(0, _B_PER_W // 16, shift, 0)

    for c in range(_NCHUNK):
        sl = pl.ds(c * _CHUNK, _CHUNK)
        cu = pltpu.async_copy(utab_hbm.at[uq_v.at[sl]], ubuf, sem)
        ci = pltpu.async_copy(itab_hbm.at[iq_v.at[sl]], ibuf, sem)
        cu.wait()
        ci.wait()
        pltpu.sync_copy(ubuf, u_out.at[pl.ds(base + c * _CHUNK, _CHUNK)])
        pltpu.sync_copy(ibuf, i_out.at[pl.ds(base + c * _CHUNK, _CHUNK)])


def _sc_gather(user_ids, item_ids, utab4, itab4):
    mesh = plsc.VectorSubcoreMesh(
        core_axis_name="c", subcore_axis_name="s",
        num_cores=_NC, num_subcores=_NS)
    f = pl.kernel(
        _sc_gather_body,
        out_type=(
            jax.ShapeDtypeStruct((BATCH, _GW), jnp.float32),
            jax.ShapeDtypeStruct((BATCH, _GW), jnp.float32),
        ),
        mesh=mesh,
        scratch_types=(
            pltpu.VMEM((_B_PER_W,), jnp.int32),
            pltpu.VMEM((_B_PER_W,), jnp.int32),
            pltpu.VMEM((_CHUNK, _GW), jnp.float32),
            pltpu.VMEM((_CHUNK, _GW), jnp.float32),
            pltpu.SemaphoreType.DMA,
        ),
        compiler_params=pltpu.CompilerParams(use_tc_tiling_on_sc=True),
    )
    return f(user_ids, item_ids, utab4, itab4)


_MLP_BLK = 2048


def _mlp_body(u4_ref, i4_ref, uid_ref, iid_ref,
              w1a_ref, w1b_ref, b1_ref, w2_ref, b2_ref,
              w3_ref, b3_ref, w4_ref, b4_ref, out_ref):
    f32 = jnp.float32
    su = lax.bitwise_and(uid_ref[...], 3)
    si = lax.bitwise_and(iid_ref[...], 3)
    u4 = u4_ref[...]
    i4 = i4_ref[...]
    u = jnp.zeros((_MLP_BLK, EMBED_DIM), f32)
    i = jnp.zeros((_MLP_BLK, EMBED_DIM), f32)
    for s in range(_GRP):
        blk = slice(s * EMBED_DIM, (s + 1) * EMBED_DIM)
        u = u + jnp.where(su == s, u4[:, blk], 0.0)
        i = i + jnp.where(si == s, i4[:, blk], 0.0)
    h = (jnp.dot(u, w1a_ref[...], preferred_element_type=f32)
         + jnp.dot(i, w1b_ref[...], preferred_element_type=f32)
         + b1_ref[...])
    h = jnp.maximum(h, 0.0)
    h = jnp.dot(h, w2_ref[...], preferred_element_type=f32) + b2_ref[...]
    h = jnp.maximum(h, 0.0)
    h = jnp.dot(h, w3_ref[...], preferred_element_type=f32) + b3_ref[...]
    h = jnp.maximum(h, 0.0)
    z = jnp.dot(h, w4_ref[...], preferred_element_type=f32) + b4_ref[...]
    out_ref[...] = jax.nn.sigmoid(z)


def _mlp(u4, i4, user_ids, item_ids, W1, b1, W2, b2, W3, b3, W4, b4):
    w1a = W1[:EMBED_DIM]
    w1b = W1[EMBED_DIM:]
    grid = BATCH // _MLP_BLK
    full = lambda a: pl.BlockSpec(a.shape, lambda b: (0,) * a.ndim)
    out = pl.pallas_call(
        _mlp_body,
        grid=(grid,),
        in_specs=[
            pl.BlockSpec((_MLP_BLK, _GW), lambda b: (b, 0)),
            pl.BlockSpec((_MLP_BLK, _GW), lambda b: (b, 0)),
            pl.BlockSpec((_MLP_BLK, 1), lambda b: (b, 0)),
            pl.BlockSpec((_MLP_BLK, 1), lambda b: (b, 0)),
            full(w1a), full(w1b),
            pl.BlockSpec((1, 64), lambda b: (0, 0)),
            full(W2),
            pl.BlockSpec((1, 32), lambda b: (0, 0)),
            full(W3),
            pl.BlockSpec((1, 16), lambda b: (0, 0)),
            full(W4),
            pl.BlockSpec((1, 1), lambda b: (0, 0)),
        ],
        out_specs=pl.BlockSpec((_MLP_BLK, 1), lambda b: (b, 0)),
        out_shape=jax.ShapeDtypeStruct((BATCH, 1), jnp.float32),
    )(u4, i4, user_ids.reshape(BATCH, 1), item_ids.reshape(BATCH, 1),
      w1a, w1b, b1.reshape(1, 64), W2, b2.reshape(1, 32),
      W3, b3.reshape(1, 16), W4, b4.reshape(1, 1))
    return out[:, 0]


def kernel(user_ids, item_ids, user_table, item_table,
           W1, b1, W2, b2, W3, b3, W4, b4):
    utab4 = user_table.reshape(-1, _GW)
    itab4 = item_table.reshape(-1, _GW)
    u4, i4 = _sc_gather(user_ids, item_ids, utab4, itab4)
    return _mlp(u4, i4, user_ids, item_ids, W1, b1, W2, b2, W3, b3, W4, b4)


# speedup vs baseline: 1.8268x; 1.6292x over previous
import jax
import jax.numpy as jnp
from jax import lax
from jax.experimental import pallas as pl
from jax.experimental.pallas import tpu as pltpu
from jax.experimental.pallas import tpu_sc as plsc

BATCH = 16384
EMBED_DIM = 32
_GW = 4 * EMBED_DIM

_UQ_LOG = 18
_IQ_LOG = 15
_UQ = 1 << _UQ_LOG
_IQ = 1 << _IQ_LOG

_NC = 2
_NS = 16
_NW = _NC * _NS
_B_PER_W = BATCH // _NW
_CHUNK = 128
_NCHUNK = _B_PER_W // _CHUNK

_PRE_BLK = 8192


def _prepass_body(x0, x1, x2, x3, o_ref):
    o_ref[...] = jnp.concatenate(
        [x0[...].T, x1[...].T, x2[...].T, x3[...].T], axis=1)


def _prepass(tabT, q):
    nblk = q // _PRE_BLK
    n = tabT.shape[1]
    max_blk = (n - 1) // _PRE_BLK

    def make_im(s):
        return lambda b: (0, jnp.minimum(s * nblk + b, max_blk))

    return pl.pallas_call(
        _prepass_body,
        grid=(nblk,),
        in_specs=[
            pl.BlockSpec((EMBED_DIM, _PRE_BLK), make_im(s)) for s in range(4)
        ],
        out_specs=pl.BlockSpec((_PRE_BLK, _GW), lambda b: (b, 0)),
        out_shape=jax.ShapeDtypeStruct((q, _GW), jnp.float32),
    )(tabT, tabT, tabT, tabT)


def _sc_gather_body(uids_hbm, iids_hbm, utab_hbm, itab_hbm,
                    u_out, i_out, uq_v, iq_v, ubuf, ibuf, sem):
    wid = lax.axis_index("s") * _NC + lax.axis_index("c")
    base = wid * _B_PER_W
    pltpu.sync_copy(uids_hbm.at[pl.ds(base, _B_PER_W)], uq_v)
    pltpu.sync_copy(iids_hbm.at[pl.ds(base, _B_PER_W)], iq_v)

    def toq(g, carry):
        sl = pl.ds(g * 16, 16)
        uq_v[sl] = lax.bitwise_and(uq_v[sl], _UQ - 1)
        iq_v[sl] = lax.bitwise_and(iq_v[sl], _IQ - 1)
        return carry
    lax.fori_loop(0, _B_PER_W // 16, toq, 0)

    for c in range(_NCHUNK):
        sl = pl.ds(c * _CHUNK, _CHUNK)
        cu = pltpu.async_copy(utab_hbm.at[uq_v.at[sl]], ubuf, sem)
        ci = pltpu.async_copy(itab_hbm.at[iq_v.at[sl]], ibuf, sem)
        cu.wait()
        ci.wait()
        pltpu.sync_copy(ubuf, u_out.at[pl.ds(base + c * _CHUNK, _CHUNK)])
        pltpu.sync_copy(ibuf, i_out.at[pl.ds(base + c * _CHUNK, _CHUNK)])


def _sc_gather(user_ids, item_ids, utab4, itab4):
    mesh = plsc.VectorSubcoreMesh(
        core_axis_name="c", subcore_axis_name="s",
        num_cores=_NC, num_subcores=_NS)
    f = pl.kernel(
        _sc_gather_body,
        out_type=(
            jax.ShapeDtypeStruct((BATCH, _GW), jnp.float32),
            jax.ShapeDtypeStruct((BATCH, _GW), jnp.float32),
        ),
        mesh=mesh,
        scratch_types=(
            pltpu.VMEM((_B_PER_W,), jnp.int32),
            pltpu.VMEM((_B_PER_W,), jnp.int32),
            pltpu.VMEM((_CHUNK, _GW), jnp.float32),
            pltpu.VMEM((_CHUNK, _GW), jnp.float32),
            pltpu.SemaphoreType.DMA,
        ),
        compiler_params=pltpu.CompilerParams(use_tc_tiling_on_sc=True),
    )
    return f(user_ids, item_ids, utab4, itab4)


_MLP_BLK = 2048


def _mlp_body(u4_ref, i4_ref, uid_ref, iid_ref,
              w1a_ref, w1b_ref, b1_ref, w2_ref, b2_ref,
              w3_ref, b3_ref, w4_ref, b4_ref, out_ref):
    f32 = jnp.float32
    su = lax.shift_right_logical(uid_ref[...], _UQ_LOG)
    si = lax.shift_right_logical(iid_ref[...], _IQ_LOG)
    u4 = u4_ref[...]
    i4 = i4_ref[...]
    u = jnp.zeros((_MLP_BLK, EMBED_DIM), f32)
    i = jnp.zeros((_MLP_BLK, EMBED_DIM), f32)
    for s in range(4):
        blk = slice(s * EMBED_DIM, (s + 1) * EMBED_DIM)
        u = u + jnp.where(su == s, u4[:, blk], 0.0)
        i = i + jnp.where(si == s, i4[:, blk], 0.0)
    h = (jnp.dot(u, w1a_ref[...], preferred_element_type=f32)
         + jnp.dot(i, w1b_ref[...], preferred_element_type=f32)
         + b1_ref[...])
    h = jnp.maximum(h, 0.0)
    h = jnp.dot(h, w2_ref[...], preferred_element_type=f32) + b2_ref[...]
    h = jnp.maximum(h, 0.0)
    h = jnp.dot(h, w3_ref[...], preferred_element_type=f32) + b3_ref[...]
    h = jnp.maximum(h, 0.0)
    z = jnp.dot(h, w4_ref[...], preferred_element_type=f32) + b4_ref[...]
    out_ref[...] = jax.nn.sigmoid(z)


def _mlp(u4, i4, user_ids, item_ids, W1, b1, W2, b2, W3, b3, W4, b4):
    w1a = W1[:EMBED_DIM]
    w1b = W1[EMBED_DIM:]
    grid = BATCH // _MLP_BLK
    full = lambda a: pl.BlockSpec(a.shape, lambda b: (0,) * a.ndim)
    out = pl.pallas_call(
        _mlp_body,
        grid=(grid,),
        in_specs=[
            pl.BlockSpec((_MLP_BLK, _GW), lambda b: (b, 0)),
            pl.BlockSpec((_MLP_BLK, _GW), lambda b: (b, 0)),
            pl.BlockSpec((_MLP_BLK, 1), lambda b: (b, 0)),
            pl.BlockSpec((_MLP_BLK, 1), lambda b: (b, 0)),
            full(w1a), full(w1b),
            pl.BlockSpec((1, 64), lambda b: (0, 0)),
            full(W2),
            pl.BlockSpec((1, 32), lambda b: (0, 0)),
            full(W3),
            pl.BlockSpec((1, 16), lambda b: (0, 0)),
            full(W4),
            pl.BlockSpec((1, 1), lambda b: (0, 0)),
        ],
        out_specs=pl.BlockSpec((_MLP_BLK, 1), lambda b: (b, 0)),
        out_shape=jax.ShapeDtypeStruct((BATCH, 1), jnp.float32),
    )(u4, i4, user_ids.reshape(BATCH, 1), item_ids.reshape(BATCH, 1),
      w1a, w1b, b1.reshape(1, 64), W2, b2.reshape(1, 32),
      W3, b3.reshape(1, 16), W4, b4.reshape(1, 1))
    return out[:, 0]


def kernel(user_ids, item_ids, user_table, item_table,
           W1, b1, W2, b2, W3, b3, W4, b4):
    utab4 = _prepass(user_table.T, _UQ)
    itab4 = _prepass(item_table.T, _IQ)
    u4, i4 = _sc_gather(user_ids, item_ids, utab4, itab4)
    return _mlp(u4, i4, user_ids, item_ids, W1, b1, W2, b2, W3, b3, W4, b4)
